# baseline (speedup 1.0000x reference)
import functools

import jax
import jax.numpy as jnp
from jax import lax
from jax.experimental import pallas as pl
from jax.experimental.pallas import tpu as pltpu
from jax.experimental.pallas import tpu_sc as plsc


def _emb_kernel(S, D, V, B, NC, NW):
    DT = 8
    CB = 2048
    n_units = S * (D // DT) * (B // CB) // NW
    upr = (D // DT) * (B // CB)
    mesh = plsc.VectorSubcoreMesh(core_axis_name="c", subcore_axis_name="s")

    @functools.partial(
        pl.kernel,
        mesh=mesh,
        out_type=jax.ShapeDtypeStruct((S, D, B), jnp.float32),
        scratch_types=[
            pltpu.VMEM((V * D,), jnp.float32),
            pltpu.VMEM((CB,), jnp.int32),
            pltpu.VMEM((CB,), jnp.int32),
            pltpu.VMEM((DT, CB), jnp.float32),
            pltpu.VMEM((DT, CB), jnp.float32),
            pltpu.SemaphoreType.DMA,
            pltpu.SemaphoreType.DMA,
            pltpu.SemaphoreType.DMA,
            pltpu.SemaphoreType.DMA,
        ],
        compiler_params=pltpu.CompilerParams(needs_layout_passes=False),
    )
    def k(wt_hbm, xt_hbm, out_hbm, wt_v, idx0, idx1, buf0, buf1,
          isem0, isem1, wsem0, wsem1):
        idxs = (idx0, idx1)
        bufs = (buf0, buf1)
        isems = (isem0, isem1)
        wsems = (wsem0, wsem1)
        wid = lax.axis_index("s") * NC + lax.axis_index("c")

        def coords(u):
            gid = u * NW + wid
            s = gid // upr
            r = gid % upr
            return s, (r // 2) * DT, (r % 2) * CB

        def gather_unit(u_s, u_d0, u_b0, idx_v, buf, wsem):
            prev = None
            for j in range(CB // 16):
                cvec = idx_v[pl.ds(j * 16, 16)]
                cur = [plsc.load_gather(wt_v, [cvec + u_d0 * V + dl * V])
                       for dl in range(DT)]
                if prev is not None:
                    pj, pvals = prev
                    for dl in range(DT):
                        buf[dl, pl.ds(pj * 16, 16)] = pvals[dl]
                prev = (j, cur)
            pj, pvals = prev
            for dl in range(DT):
                buf[dl, pl.ds(pj * 16, 16)] = pvals[dl]
            pltpu.async_copy(
                buf, out_hbm.at[u_s, pl.ds(u_d0, DT), pl.ds(u_b0, CB)],
                wsem)

        pltpu.sync_copy(wt_hbm, wt_v)
        for h in range(2):
            s, _, b0 = coords(h)
            pltpu.async_copy(xt_hbm.at[s, pl.ds(b0, CB)], idxs[h], isems[h])

        def body(i, carry):
            for h in range(2):
                u = 2 * i + h
                gid = u * NW + wid
                s = gid // upr
                r = gid % upr
                d0 = (r // 2) * DT
                b0 = (r % 2) * CB
                buf = bufs[h]
                pltpu.make_async_copy(
                    xt_hbm.at[s, pl.ds(b0, CB)], idxs[h], isems[h]).wait()

                @pl.when(i > 0)
                def _():
                    pltpu.make_async_copy(
                        buf,
                        out_hbm.at[s, pl.ds(d0, DT), pl.ds(b0, CB)],
                        wsems[h]).wait()

                gather_unit(s, d0, b0, idxs[h], buf, wsems[h])

                @pl.when(u + 2 < n_units)
                def _():
                    gid2 = gid + 2 * NW
                    s2 = gid2 // upr
                    b02 = (gid2 % 2) * CB
                    pltpu.async_copy(
                        xt_hbm.at[s2, pl.ds(b02, CB)], idxs[h], isems[h])
            return carry

        lax.fori_loop(0, n_units // 2, body, 0)

        for u in range(2 * (n_units // 2), n_units):
            h = u % 2
            s, d0, b0 = coords(u)
            pltpu.make_async_copy(
                xt_hbm.at[s, pl.ds(b0, CB)], idxs[h], isems[h]).wait()
            pltpu.make_async_copy(
                bufs[h], out_hbm.at[s, pl.ds(d0, DT), pl.ds(b0, CB)],
                wsems[h]).wait()
            gather_unit(s, d0, b0, idxs[h], bufs[h], wsems[h])
        for h in range(2):
            pltpu.make_async_copy(
                bufs[h], out_hbm.at[0, pl.ds(0, DT), pl.ds(0, CB)],
                wsems[h]).wait()

    return k


def kernel(x, weights):
    Bdim, S = x.shape
    V, D = weights.shape
    info = plsc.get_sparse_core_info()
    NC, NS = info.num_cores, info.num_subcores
    NW = NC * NS
    wt_flat = weights.astype(jnp.float32).T.reshape(V * D)
    xt = x.astype(jnp.int32).T
    k = _emb_kernel(S, D, V, Bdim, NC, NW)
    out_t = k(wt_flat, xt)
    return jnp.transpose(out_t, (2, 0, 1))


# speedup vs baseline: 1.3456x; 1.3456x over previous
import functools

import jax
import jax.numpy as jnp
from jax import lax
from jax.experimental import pallas as pl
from jax.experimental.pallas import tpu as pltpu
from jax.experimental.pallas import tpu_sc as plsc


def _emb_kernel(S, D, V, B, NC, NW):
    DT = 8
    CB = 2048
    n_units = S * (D // DT) * (B // CB) // NW
    upr = (D // DT) * (B // CB)
    mesh = plsc.VectorSubcoreMesh(core_axis_name="c", subcore_axis_name="s")

    @functools.partial(
        pl.kernel,
        mesh=mesh,
        out_type=jax.ShapeDtypeStruct((S, D, B), jnp.float32),
        scratch_types=[
            pltpu.VMEM((V * D,), jnp.float32),
            pltpu.VMEM((CB,), jnp.int32),
            pltpu.VMEM((CB,), jnp.int32),
            pltpu.VMEM((DT, CB), jnp.float32),
            pltpu.VMEM((DT, CB), jnp.float32),
            pltpu.SemaphoreType.DMA,
            pltpu.SemaphoreType.DMA,
            pltpu.SemaphoreType.DMA,
            pltpu.SemaphoreType.DMA,
        ],
        compiler_params=pltpu.CompilerParams(needs_layout_passes=False),
    )
    def k(wt_hbm, xt_hbm, out_hbm, wt_v, idx0, idx1, buf0, buf1,
          isem0, isem1, wsem0, wsem1):
        idxs = (idx0, idx1)
        bufs = (buf0, buf1)
        isems = (isem0, isem1)
        wsems = (wsem0, wsem1)
        wid = lax.axis_index("s") * NC + lax.axis_index("c")

        def coords(u):
            gid = u * NW + wid
            s = gid // upr
            r = gid % upr
            return s, (r // 2) * DT, (r % 2) * CB

        def gather_unit(u_s, u_d0, u_b0, idx_v, buf, wsem):
            prev = None
            base = u_d0 * V
            for jj in range(0, CB // 16, 8):
                cvecs8 = [idx_v[pl.ds((jj + t) * 16, 16)] + base
                          for t in range(8)]
                for tp in range(0, 8, 2):
                    j = jj + tp
                    cur = []
                    for n in range(2 * DT):
                        cur.append(plsc.load_gather(
                            wt_v, [cvecs8[tp + (n // DT)] + (n % DT) * V]))
                        if prev is not None:
                            pj, pvals = prev
                            buf[n % DT,
                                pl.ds((pj + n // DT) * 16, 16)] = pvals[n]
                    prev = (j, cur)
            pj, pvals = prev
            for n in range(2 * DT):
                buf[n % DT, pl.ds((pj + n // DT) * 16, 16)] = pvals[n]
            pltpu.async_copy(
                buf, out_hbm.at[u_s, pl.ds(u_d0, DT), pl.ds(u_b0, CB)],
                wsem)

        pltpu.sync_copy(wt_hbm, wt_v)
        for h in range(2):
            s, _, b0 = coords(h)
            pltpu.async_copy(xt_hbm.at[s, pl.ds(b0, CB)], idxs[h], isems[h])

        def body(i, carry):
            for h in range(2):
                u = 2 * i + h
                gid = u * NW + wid
                s = gid // upr
                r = gid % upr
                d0 = (r // 2) * DT
                b0 = (r % 2) * CB
                buf = bufs[h]
                pltpu.make_async_copy(
                    xt_hbm.at[s, pl.ds(b0, CB)], idxs[h], isems[h]).wait()

                @pl.when(i > 0)
                def _():
                    pltpu.make_async_copy(
                        buf,
                        out_hbm.at[s, pl.ds(d0, DT), pl.ds(b0, CB)],
                        wsems[h]).wait()

                gather_unit(s, d0, b0, idxs[h], buf, wsems[h])

                @pl.when(u + 2 < n_units)
                def _():
                    gid2 = gid + 2 * NW
                    s2 = gid2 // upr
                    b02 = (gid2 % 2) * CB
                    pltpu.async_copy(
                        xt_hbm.at[s2, pl.ds(b02, CB)], idxs[h], isems[h])
            return carry

        lax.fori_loop(0, n_units // 2, body, 0)

        for u in range(2 * (n_units // 2), n_units):
            h = u % 2
            s, d0, b0 = coords(u)
            pltpu.make_async_copy(
                xt_hbm.at[s, pl.ds(b0, CB)], idxs[h], isems[h]).wait()
            pltpu.make_async_copy(
                bufs[h], out_hbm.at[s, pl.ds(d0, DT), pl.ds(b0, CB)],
                wsems[h]).wait()
            gather_unit(s, d0, b0, idxs[h], bufs[h], wsems[h])
        for h in range(2):
            pltpu.make_async_copy(
                bufs[h], out_hbm.at[0, pl.ds(0, DT), pl.ds(0, CB)],
                wsems[h]).wait()

    return k


def kernel(x, weights):
    Bdim, S = x.shape
    V, D = weights.shape
    info = plsc.get_sparse_core_info()
    NC, NS = info.num_cores, info.num_subcores
    NW = NC * NS
    wt_flat = weights.astype(jnp.float32).T.reshape(V * D)
    xt = x.astype(jnp.int32).T
    k = _emb_kernel(S, D, V, Bdim, NC, NW)
    out_t = k(wt_flat, xt)
    return jnp.transpose(out_t, (2, 0, 1))


# speedup vs baseline: 1.8269x; 1.3577x over previous
import functools

import jax
import jax.numpy as jnp
from jax import lax
from jax.experimental import pallas as pl
from jax.experimental.pallas import tpu as pltpu
from jax.experimental.pallas import tpu_sc as plsc


def _emb_kernel(S, D, V, B, NC, NW):
    BLK = B // NW
    mesh = plsc.VectorSubcoreMesh(core_axis_name="c", subcore_axis_name="s")

    @functools.partial(
        pl.kernel,
        mesh=mesh,
        out_type=jax.ShapeDtypeStruct((S, D, B), jnp.float32),
        scratch_types=[
            pltpu.VMEM((V * D,), jnp.float32),
            pltpu.VMEM((BLK,), jnp.int32),
            pltpu.VMEM((BLK,), jnp.int32),
            pltpu.VMEM((D, BLK), jnp.float32),
            pltpu.VMEM((D, BLK), jnp.float32),
            pltpu.SemaphoreType.DMA,
            pltpu.SemaphoreType.DMA,
            pltpu.SemaphoreType.DMA,
            pltpu.SemaphoreType.DMA,
        ],
        compiler_params=pltpu.CompilerParams(needs_layout_passes=False),
    )
    def k(wt_hbm, xt_hbm, out_hbm, wt_v, idx0, idx1, buf0, buf1,
          isem0, isem1, wsem0, wsem1):
        wid = lax.axis_index("s") * NC + lax.axis_index("c")
        col0 = wid * BLK
        idxs = (idx0, idx1)
        bufs = (buf0, buf1)
        isems = (isem0, isem1)
        wsems = (wsem0, wsem1)

        pltpu.sync_copy(wt_hbm, wt_v)
        for h in range(2):
            pltpu.async_copy(xt_hbm.at[h, pl.ds(col0, BLK)], idxs[h],
                             isems[h])

        def body(i, carry):
            for h in range(2):
                s = 2 * i + h
                idx_v, buf = idxs[h], bufs[h]
                pltpu.make_async_copy(
                    xt_hbm.at[s, pl.ds(col0, BLK)], idx_v, isems[h]).wait()
                cvecs = [idx_v[pl.ds(g * 16, 16)] for g in range(BLK // 16)]

                @pl.when(i < (S // 2) - 1)
                def _():
                    pltpu.async_copy(
                        xt_hbm.at[s + 2, pl.ds(col0, BLK)], idx_v, isems[h])

                @pl.when(i > 0)
                def _():
                    pltpu.make_async_copy(
                        buf, out_hbm.at[s, :, pl.ds(col0, BLK)],
                        wsems[h]).wait()

                blocks = [(g, d0) for g in range(BLK // 16)
                          for d0 in range(0, D, 16)]
                prev = None
                for g, d0 in blocks:
                    cvec = cvecs[g]
                    cur = []
                    for u in range(16):
                        cur.append(
                            plsc.load_gather(wt_v, [cvec + (d0 + u) * V]))
                        if prev is not None:
                            pg, pd0, pvals = prev
                            buf[pd0 + u, pl.ds(pg * 16, 16)] = pvals[u]
                    prev = (g, d0, cur)
                pg, pd0, pvals = prev
                for u in range(16):
                    buf[pd0 + u, pl.ds(pg * 16, 16)] = pvals[u]
                pltpu.async_copy(
                    buf, out_hbm.at[s, :, pl.ds(col0, BLK)], wsems[h])
            return carry

        lax.fori_loop(0, S // 2, body, 0)
        for h in range(2):
            s = S - 2 + h
            pltpu.make_async_copy(
                bufs[h], out_hbm.at[s, :, pl.ds(col0, BLK)], wsems[h]).wait()

    return k


def kernel(x, weights):
    Bdim, S = x.shape
    V, D = weights.shape
    info = plsc.get_sparse_core_info()
    NC, NS = info.num_cores, info.num_subcores
    NW = NC * NS
    wt_flat = weights.astype(jnp.float32).T.reshape(V * D)
    xt = x.astype(jnp.int32).T
    k = _emb_kernel(S, D, V, Bdim, NC, NW)
    out_t = k(wt_flat, xt)
    return jnp.transpose(out_t, (2, 0, 1))
